# baseline (speedup 1.0000x reference)
import functools

import jax
import jax.numpy as jnp
from jax import lax
from jax.experimental import pallas as pl
from jax.experimental.pallas import tpu as pltpu
from jax.experimental.pallas import tpu_sc as plsc

N = 100000
E = 3200000

NC = 2
NS = 16
NW = NC * NS
K = 128
C = 8
NRB = 8
GAHEAD = 3

NPAD = 100352
RPS = NPAD // NS
NBLK = 792
EPAD = NW * NBLK * K
TOTBLK = EPAD // K
NCHUNK = NBLK // C
SDEPTH = 5

F = 16
NP = NPAD // 8

_mesh = plsc.VectorSubcoreMesh(core_axis_name="c", subcore_axis_name="s")
_sc_params = pltpu.CompilerParams(use_tc_tiling_on_sc=False)


def _make_degree(nrows, w):
  rps = nrows // NS

  @functools.partial(
      pl.kernel,
      out_type=jax.ShapeDtypeStruct((NC * nrows, w), jnp.float32),
      mesh=_mesh,
      scratch_types=[
          pltpu.VMEM((3, C, K), jnp.int32),
          pltpu.VMEM((K, w), jnp.float32),
          pltpu.VMEM_SHARED((nrows, w), jnp.float32),
          pltpu.SemaphoreType.DMA,
          pltpu.SemaphoreType.DMA,
      ],
      compiler_params=_sc_params,
  )
  def _sc_degree(dst_hbm, ones_hbm, zero_hbm, out_hbm, didx, ones_v, acc,
                 isem, ssem):
    cid = lax.axis_index("c")
    sid = lax.axis_index("s")
    wid = sid * NC + cid
    r0 = sid * rps
    pltpu.sync_copy(ones_hbm, ones_v)
    pltpu.sync_copy(zero_hbm.at[pl.ds(r0, rps)], acc.at[pl.ds(r0, rps)])
    plsc.subcore_barrier()

    bb = wid * NBLK
    pltpu.sync_copy(dst_hbm.at[pl.ds(bb, C)], didx.at[0])

    def swait():
        pltpu.make_async_copy(ones_v, acc.at[didx.at[0, 0]], ssem).wait()

    @pl.loop(0, NCHUNK // 3)
    def _(t):
        for q in range(3):
            c = t * 3 + q

            @pl.when(c < NCHUNK - 1)
            def _():
                nb = bb + (c + 1) * C
                pltpu.async_copy(dst_hbm.at[pl.ds(nb, C)],
                                 didx.at[(q + 1) % 3], isem)

            for j in range(C):
                if q == 0 and j < SDEPTH:
                    @pl.when(c > 0)
                    def _():
                        swait()
                else:
                    swait()
                pltpu.async_copy(ones_v, acc.at[didx.at[q, j]], ssem,
                                 add=True)

            @pl.when(c < NCHUNK - 1)
            def _():
                pltpu.make_async_copy(
                    dst_hbm.at[pl.ds(bb, C)], didx.at[(q + 1) % 3],
                    isem).wait()

    for _k in range(SDEPTH):
        swait()
    plsc.subcore_barrier()
    pltpu.sync_copy(acc.at[pl.ds(r0, rps)],
                    out_hbm.at[pl.ds(cid * nrows + r0, rps)])

  return _sc_degree


def _make_propagate(nrows, w):
  rps = nrows // NS

  @functools.partial(
      pl.kernel,
      out_type=jax.ShapeDtypeStruct((NC * nrows, w), jnp.float32),
      mesh=_mesh,
      scratch_types=[
          pltpu.VMEM((3, C, K), jnp.int32),
          pltpu.VMEM((3, C, K), jnp.int32),
      ] + [pltpu.VMEM((K, w), jnp.float32)] * NRB + [
          pltpu.VMEM_SHARED((nrows, w), jnp.float32),
          pltpu.SemaphoreType.DMA,
          pltpu.SemaphoreType.DMA,
          pltpu.SemaphoreType.DMA,
      ],
      compiler_params=_sc_params,
  )
  def _sc_propagate(g_hbm, src_hbm, dst_hbm, zero_hbm, out_hbm,
                    sidx, didx, rows0, rows1, rows2, rows3, rows4, rows5,
                    rows6, rows7, acc, gsem, isem, ssem):
    cid = lax.axis_index("c")
    sid = lax.axis_index("s")
    wid = sid * NC + cid
    r0 = sid * rps
    pltpu.sync_copy(zero_hbm.at[pl.ds(r0, rps)], acc.at[pl.ds(r0, rps)])
    plsc.subcore_barrier()

    bb = wid * NBLK
    pltpu.sync_copy(src_hbm.at[pl.ds(bb, C)], sidx.at[0])
    pltpu.sync_copy(dst_hbm.at[pl.ds(bb, C)], didx.at[0])
    rbufs = (rows0, rows1, rows2, rows3, rows4, rows5, rows6, rows7)
    for j in range(GAHEAD):
        pltpu.async_copy(g_hbm.at[sidx.at[0, j]], rbufs[j], gsem)

    def swait(buf):
        pltpu.make_async_copy(buf, acc.at[didx.at[0, 0]], ssem).wait()

    @pl.loop(0, NCHUNK // 3)
    def _(t):
        for q in range(3):
            c = t * 3 + q

            @pl.when(c < NCHUNK - 1)
            def _():
                nb = bb + (c + 1) * C
                pltpu.async_copy(src_hbm.at[pl.ds(nb, C)],
                                 sidx.at[(q + 1) % 3], isem)
                pltpu.async_copy(dst_hbm.at[pl.ds(nb, C)],
                                 didx.at[(q + 1) % 3], isem)

            for j in range(C):
                fbuf = rbufs[(j + GAHEAD) % NRB]
                if q == 0 and j < NRB - GAHEAD:
                    @pl.when(c > 0)
                    def _():
                        swait(fbuf)
                else:
                    swait(fbuf)
                ahead = j + GAHEAD
                if ahead < C:
                    pltpu.async_copy(g_hbm.at[sidx.at[q, ahead]], fbuf, gsem)
                else:
                    @pl.when(c < NCHUNK - 1)
                    def _():
                        if ahead == C:
                            pltpu.make_async_copy(
                                src_hbm.at[pl.ds(bb, C)],
                                sidx.at[(q + 1) % 3], isem).wait()
                            pltpu.make_async_copy(
                                dst_hbm.at[pl.ds(bb, C)],
                                didx.at[(q + 1) % 3], isem).wait()
                        pltpu.async_copy(
                            g_hbm.at[sidx.at[(q + 1) % 3, ahead - C]], fbuf,
                            gsem)
                cur = rbufs[j]
                pltpu.make_async_copy(g_hbm.at[sidx.at[q, j]], cur,
                                      gsem).wait()
                pltpu.async_copy(cur, acc.at[didx.at[q, j]], ssem, add=True)

    for k in range(NRB - GAHEAD):
        swait(rbufs[(GAHEAD + k) % NRB])
    plsc.subcore_barrier()
    pltpu.sync_copy(acc.at[pl.ds(r0, rps)],
                    out_hbm.at[pl.ds(cid * nrows + r0, rps)])

  return _sc_propagate


_sc_degree8 = _make_degree(2 * NPAD, 8)
_sc_prop16 = _make_propagate(NPAD, F)
_sc_prop8 = _make_propagate(2 * NPAD, 8)


NB = 7
BP = NP // NB
XW = 8 * 21


def _tc1_body(x_ref, c0_ref, c1_ref, bdw1_ref, b1t_ref, bdwc1_ref,
              g1_ref, dinv_ref):
    c = c0_ref[...] + c1_ref[...]
    c = c + jnp.concatenate([jnp.zeros((BP, 8), jnp.float32), c[:, :-8]],
                            axis=1)
    dinv = lax.rsqrt(c + 1.0)
    bdm1 = jnp.dot(bdw1_ref[...], bdwc1_ref[...],
                   preferred_element_type=jnp.float32)
    c1t = jnp.dot(b1t_ref[...], bdwc1_ref[...],
                  preferred_element_type=jnp.float32)
    g1_ref[...] = (jnp.dot(jnp.trunc(x_ref[...]), bdm1,
                           preferred_element_type=jnp.float32) + c1t) * dinv
    dinv_ref[...] = dinv


def _tc1(x_pk, cntp, bd_fc1w, fc1b_t, bd_w1):
    return pl.pallas_call(
        _tc1_body,
        grid=(NB,),
        in_specs=[
            pl.BlockSpec((BP, XW), lambda i: (i, 0)),
            pl.BlockSpec((BP, 128), lambda i: (i, 0)),
            pl.BlockSpec((BP, 128), lambda i: (NB + i, 0)),
            pl.BlockSpec((XW, 8 * 24), lambda i: (0, 0)),
            pl.BlockSpec((1, 8 * 24), lambda i: (0, 0)),
            pl.BlockSpec((8 * 24, 128), lambda i: (0, 0)),
        ],
        out_specs=[
            pl.BlockSpec((BP, 128), lambda i: (i, 0)),
            pl.BlockSpec((BP, 128), lambda i: (i, 0)),
        ],
        out_shape=[
            jax.ShapeDtypeStruct((NP, 128), jnp.float32),
            jax.ShapeDtypeStruct((NP, 128), jnp.float32),
        ],
    )(x_pk, cntp, cntp, bd_fc1w, fc1b_t, bd_w1)


def _tc2_body(a0_ref, a1_ref, g1_ref, dinv_ref, b1t_ref, bdw2_ref, g2_ref):
    d = dinv_ref[...]
    h1 = d * (a0_ref[...] + a1_ref[...] + g1_ref[...]) + b1t_ref[...]
    h1 = jnp.maximum(h1, 0.0)
    g2_ref[...] = jnp.dot(h1, bdw2_ref[...],
                          preferred_element_type=jnp.float32) * d


def _tc2(agg1p, g1p, dinvp, b1_t, bd_w2):
    return pl.pallas_call(
        _tc2_body,
        grid=(NB,),
        in_specs=[
            pl.BlockSpec((BP, 128), lambda i: (i, 0)),
            pl.BlockSpec((BP, 128), lambda i: (NB + i, 0)),
            pl.BlockSpec((BP, 128), lambda i: (i, 0)),
            pl.BlockSpec((BP, 128), lambda i: (i, 0)),
            pl.BlockSpec((1, 128), lambda i: (0, 0)),
            pl.BlockSpec((128, 128), lambda i: (0, 0)),
        ],
        out_specs=pl.BlockSpec((BP, 128), lambda i: (i, 0)),
        out_shape=jax.ShapeDtypeStruct((NP, 128), jnp.float32),
    )(agg1p, agg1p, g1p, dinvp, b1_t, bd_w2)


def _tc3_body(a0_ref, a1_ref, g2_ref, dinv_ref, b2t_ref, out_ref):
    o = dinv_ref[...] * (a0_ref[...] + a1_ref[...] + g2_ref[...]) + b2t_ref[...]
    mask = lax.broadcasted_iota(jnp.int32, (1, 128), 1) % F < 6
    z = jnp.where(mask, o, -jnp.inf)
    m = jnp.concatenate(
        [jnp.broadcast_to(
            jnp.max(z[:, i * F:(i + 1) * F], axis=1, keepdims=True), (BP, F))
         for i in range(8)], axis=1)
    e = jnp.where(mask, jnp.exp(z - m), 0.0)
    s = jnp.concatenate(
        [jnp.broadcast_to(
            jnp.sum(e[:, i * F:(i + 1) * F], axis=1, keepdims=True), (BP, F))
         for i in range(8)], axis=1)
    out_ref[...] = o - (m + jnp.log(s))


def _tc3(agg2p, g2p, dinvp, b2_t):
    return pl.pallas_call(
        _tc3_body,
        grid=(NB,),
        in_specs=[
            pl.BlockSpec((BP, 128), lambda i: (i, 0)),
            pl.BlockSpec((BP, 128), lambda i: (NB + i, 0)),
            pl.BlockSpec((BP, 128), lambda i: (i, 0)),
            pl.BlockSpec((BP, 128), lambda i: (i, 0)),
            pl.BlockSpec((1, 128), lambda i: (0, 0)),
        ],
        out_specs=pl.BlockSpec((BP, 128), lambda i: (i, 0)),
        out_shape=jax.ShapeDtypeStruct((NP, 128), jnp.float32),
    )(agg2p, agg2p, g2p, dinvp, b2_t)


def kernel(x, edge_index, fc1_w, fc1_b, conv1_w, conv1_b, conv2_w, conv2_b):
    src = edge_index[0]
    dst = edge_index[1]
    pad_idx = (N + jnp.arange(EPAD - E, dtype=jnp.int32) % (NPAD - N))
    src_p = jnp.concatenate([src, pad_idx]).reshape(TOTBLK, K)
    dst_p = jnp.concatenate([dst, pad_idx]).reshape(TOTBLK, K)
    src2_p = src_p * 2
    dst2_p = dst_p * 2
    x_pk = jnp.pad(x, ((0, NPAD - N), (0, 0))).reshape(NP, XW)
    zeros16 = jnp.zeros((NPAD, F), jnp.float32)
    zeros8 = jnp.zeros((2 * NPAD, 8), jnp.float32)
    ones8 = jnp.ones((K, 8), jnp.float32)
    eye8 = jnp.eye(8, dtype=jnp.float32)
    bd_fc1w = jnp.kron(eye8, fc1_w)
    bd_w1 = jnp.kron(eye8, conv1_w)
    w2_pad = jnp.pad(conv2_w, ((0, 0), (0, F - 6)))
    bd_w2 = jnp.kron(eye8, w2_pad)
    fc1b_t = jnp.tile(fc1_b, 8).reshape(1, 8 * 24)
    b1_t = jnp.tile(conv1_b, 8).reshape(1, 128)
    b2_t = jnp.tile(jnp.pad(conv2_b, ((0, F - 6),)), 8).reshape(1, 128)

    cnt = _sc_degree8(dst2_p, ones8, zeros8)
    cntp = jnp.reshape(cnt, (2 * NP, 128))
    g1p, dinvp = _tc1(x_pk, cntp, bd_fc1w, fc1b_t, bd_w1)
    agg1 = _sc_prop16(jnp.reshape(g1p, (NPAD, F)), src_p, dst_p, zeros16)
    agg1p = jnp.reshape(agg1, (2 * NP, 128))
    g2p = _tc2(agg1p, g1p, dinvp, b1_t, bd_w2)
    agg2 = _sc_prop8(jnp.reshape(g2p, (2 * NPAD, 8)), src2_p, dst2_p, zeros8)
    agg2p = jnp.reshape(agg2, (2 * NP, 128))
    outp = _tc3(agg2p, g2p, dinvp, b2_t)
    return jnp.reshape(outp, (NPAD, F))[:N, :6]


# speedup vs baseline: 1.1623x; 1.1623x over previous
import functools

import jax
import jax.numpy as jnp
from jax import lax
from jax.experimental import pallas as pl
from jax.experimental.pallas import tpu as pltpu
from jax.experimental.pallas import tpu_sc as plsc

N = 100000
E = 3200000

NC = 2
NS = 16
NW = NC * NS
K = 128
C = 8
NRB = 8
GAHEAD = 4

NPAD = 100352
RPS = NPAD // NS
NBLK = 792
EPAD = NW * NBLK * K
TOTBLK = EPAD // K
NCHUNK = NBLK // C
SDEPTH = 5

F = 16
NP = NPAD // 8

_mesh = plsc.VectorSubcoreMesh(core_axis_name="c", subcore_axis_name="s")
_sc_params = pltpu.CompilerParams(use_tc_tiling_on_sc=False)


def _make_degree(nrows, w):
  rps = nrows // NS

  @functools.partial(
      pl.kernel,
      out_type=jax.ShapeDtypeStruct((NC * nrows, w), jnp.float32),
      mesh=_mesh,
      scratch_types=[
          pltpu.VMEM((3, C, K), jnp.int32),
          pltpu.VMEM((K, w), jnp.float32),
          pltpu.VMEM_SHARED((nrows, w), jnp.float32),
          pltpu.SemaphoreType.DMA,
          pltpu.SemaphoreType.DMA,
      ],
      compiler_params=_sc_params,
  )
  def _sc_degree(dst_hbm, ones_hbm, zero_hbm, out_hbm, didx, ones_v, acc,
                 isem, ssem):
    cid = lax.axis_index("c")
    sid = lax.axis_index("s")
    wid = sid * NC + cid
    r0 = sid * rps
    pltpu.sync_copy(ones_hbm, ones_v)
    pltpu.sync_copy(zero_hbm.at[pl.ds(r0, rps)], acc.at[pl.ds(r0, rps)])
    plsc.subcore_barrier()

    bb = wid * NBLK
    pltpu.sync_copy(dst_hbm.at[pl.ds(bb, C)], didx.at[0])

    def swait():
        pltpu.make_async_copy(ones_v, acc.at[didx.at[0, 0]], ssem).wait()

    @pl.loop(0, NCHUNK // 3)
    def _(t):
        for q in range(3):
            c = t * 3 + q

            @pl.when(c < NCHUNK - 1)
            def _():
                nb = bb + (c + 1) * C
                pltpu.async_copy(dst_hbm.at[pl.ds(nb, C)],
                                 didx.at[(q + 1) % 3], isem)

            for j in range(C):
                if q == 0 and j < SDEPTH:
                    @pl.when(c > 0)
                    def _():
                        swait()
                else:
                    swait()
                pltpu.async_copy(ones_v, acc.at[didx.at[q, j]], ssem,
                                 add=True)

            @pl.when(c < NCHUNK - 1)
            def _():
                pltpu.make_async_copy(
                    dst_hbm.at[pl.ds(bb, C)], didx.at[(q + 1) % 3],
                    isem).wait()

    for _k in range(SDEPTH):
        swait()
    plsc.subcore_barrier()
    pltpu.sync_copy(acc.at[pl.ds(r0, rps)],
                    out_hbm.at[pl.ds(cid * nrows + r0, rps)])

  return _sc_degree


def _make_propagate(nrows, w):
  rps = nrows // NS

  @functools.partial(
      pl.kernel,
      out_type=jax.ShapeDtypeStruct((NC * nrows, w), jnp.float32),
      mesh=_mesh,
      scratch_types=[
          pltpu.VMEM((3, C, K), jnp.int32),
          pltpu.VMEM((3, C, K), jnp.int32),
      ] + [pltpu.VMEM((K, w), jnp.float32)] * NRB + [
          pltpu.VMEM_SHARED((nrows, w), jnp.float32),
          pltpu.SemaphoreType.DMA,
          pltpu.SemaphoreType.DMA,
          pltpu.SemaphoreType.DMA,
      ],
      compiler_params=_sc_params,
  )
  def _sc_propagate(g_hbm, src_hbm, dst_hbm, zero_hbm, out_hbm,
                    sidx, didx, rows0, rows1, rows2, rows3, rows4, rows5,
                    rows6, rows7, acc, gsem, isem, ssem):
    cid = lax.axis_index("c")
    sid = lax.axis_index("s")
    wid = sid * NC + cid
    r0 = sid * rps
    pltpu.sync_copy(zero_hbm.at[pl.ds(r0, rps)], acc.at[pl.ds(r0, rps)])
    plsc.subcore_barrier()

    bb = wid * NBLK
    pltpu.sync_copy(src_hbm.at[pl.ds(bb, C)], sidx.at[0])
    pltpu.sync_copy(dst_hbm.at[pl.ds(bb, C)], didx.at[0])
    rbufs = (rows0, rows1, rows2, rows3, rows4, rows5, rows6, rows7)
    for j in range(GAHEAD):
        pltpu.async_copy(g_hbm.at[sidx.at[0, j]], rbufs[j], gsem)

    def swait(buf):
        pltpu.make_async_copy(buf, acc.at[didx.at[0, 0]], ssem).wait()

    @pl.loop(0, NCHUNK // 3)
    def _(t):
        for q in range(3):
            c = t * 3 + q

            @pl.when(c < NCHUNK - 1)
            def _():
                nb = bb + (c + 1) * C
                pltpu.async_copy(src_hbm.at[pl.ds(nb, C)],
                                 sidx.at[(q + 1) % 3], isem)
                pltpu.async_copy(dst_hbm.at[pl.ds(nb, C)],
                                 didx.at[(q + 1) % 3], isem)

            for j in range(C):
                fbuf = rbufs[(j + GAHEAD) % NRB]
                if q == 0 and j < NRB - GAHEAD:
                    @pl.when(c > 0)
                    def _():
                        swait(fbuf)
                else:
                    swait(fbuf)
                ahead = j + GAHEAD
                if ahead < C:
                    pltpu.async_copy(g_hbm.at[sidx.at[q, ahead]], fbuf, gsem)
                else:
                    @pl.when(c < NCHUNK - 1)
                    def _():
                        if ahead == C:
                            pltpu.make_async_copy(
                                src_hbm.at[pl.ds(bb, C)],
                                sidx.at[(q + 1) % 3], isem).wait()
                            pltpu.make_async_copy(
                                dst_hbm.at[pl.ds(bb, C)],
                                didx.at[(q + 1) % 3], isem).wait()
                        pltpu.async_copy(
                            g_hbm.at[sidx.at[(q + 1) % 3, ahead - C]], fbuf,
                            gsem)
                cur = rbufs[j]
                pltpu.make_async_copy(g_hbm.at[sidx.at[q, j]], cur,
                                      gsem).wait()
                pltpu.async_copy(cur, acc.at[didx.at[q, j]], ssem, add=True)

    for k in range(NRB - GAHEAD):
        swait(rbufs[(GAHEAD + k) % NRB])
    plsc.subcore_barrier()
    pltpu.sync_copy(acc.at[pl.ds(r0, rps)],
                    out_hbm.at[pl.ds(cid * nrows + r0, rps)])

  return _sc_propagate


_sc_degree16 = _make_degree(NPAD, F)
_sc_prop16 = _make_propagate(NPAD, F)


NB = 7
BP = NP // NB
XW = 8 * 21


def _tc1_body(x_ref, c0_ref, c1_ref, bdw1_ref, b1t_ref, bdwc1_ref,
              g1_ref, dinv_ref):
    dinv = lax.rsqrt(c0_ref[...] + c1_ref[...] + 1.0)
    bdm1 = jnp.dot(bdw1_ref[...], bdwc1_ref[...],
                   preferred_element_type=jnp.float32)
    c1t = jnp.dot(b1t_ref[...], bdwc1_ref[...],
                  preferred_element_type=jnp.float32)
    g1_ref[...] = (jnp.dot(jnp.trunc(x_ref[...]), bdm1,
                           preferred_element_type=jnp.float32) + c1t) * dinv
    dinv_ref[...] = dinv


def _tc1(x_pk, cntp, bd_fc1w, fc1b_t, bd_w1):
    return pl.pallas_call(
        _tc1_body,
        grid=(NB,),
        in_specs=[
            pl.BlockSpec((BP, XW), lambda i: (i, 0)),
            pl.BlockSpec((BP, 128), lambda i: (i, 0)),
            pl.BlockSpec((BP, 128), lambda i: (NB + i, 0)),
            pl.BlockSpec((XW, 8 * 24), lambda i: (0, 0)),
            pl.BlockSpec((1, 8 * 24), lambda i: (0, 0)),
            pl.BlockSpec((8 * 24, 128), lambda i: (0, 0)),
        ],
        out_specs=[
            pl.BlockSpec((BP, 128), lambda i: (i, 0)),
            pl.BlockSpec((BP, 128), lambda i: (i, 0)),
        ],
        out_shape=[
            jax.ShapeDtypeStruct((NP, 128), jnp.float32),
            jax.ShapeDtypeStruct((NP, 128), jnp.float32),
        ],
    )(x_pk, cntp, cntp, bd_fc1w, fc1b_t, bd_w1)


def _tc2_body(a0_ref, a1_ref, g1_ref, dinv_ref, b1t_ref, bdw2_ref, g2_ref):
    d = dinv_ref[...]
    h1 = d * (a0_ref[...] + a1_ref[...] + g1_ref[...]) + b1t_ref[...]
    h1 = jnp.maximum(h1, 0.0)
    g2_ref[...] = jnp.dot(h1, bdw2_ref[...],
                          preferred_element_type=jnp.float32) * d


def _tc2(agg1p, g1p, dinvp, b1_t, bd_w2):
    return pl.pallas_call(
        _tc2_body,
        grid=(NB,),
        in_specs=[
            pl.BlockSpec((BP, 128), lambda i: (i, 0)),
            pl.BlockSpec((BP, 128), lambda i: (NB + i, 0)),
            pl.BlockSpec((BP, 128), lambda i: (i, 0)),
            pl.BlockSpec((BP, 128), lambda i: (i, 0)),
            pl.BlockSpec((1, 128), lambda i: (0, 0)),
            pl.BlockSpec((128, 128), lambda i: (0, 0)),
        ],
        out_specs=pl.BlockSpec((BP, 128), lambda i: (i, 0)),
        out_shape=jax.ShapeDtypeStruct((NP, 128), jnp.float32),
    )(agg1p, agg1p, g1p, dinvp, b1_t, bd_w2)


def _tc3_body(a0_ref, a1_ref, g2_ref, dinv_ref, b2t_ref, out_ref):
    o = dinv_ref[...] * (a0_ref[...] + a1_ref[...] + g2_ref[...]) + b2t_ref[...]
    mask = lax.broadcasted_iota(jnp.int32, (1, 128), 1) % F < 6
    z = jnp.where(mask, o, -jnp.inf)
    m = jnp.concatenate(
        [jnp.broadcast_to(
            jnp.max(z[:, i * F:(i + 1) * F], axis=1, keepdims=True), (BP, F))
         for i in range(8)], axis=1)
    e = jnp.where(mask, jnp.exp(z - m), 0.0)
    s = jnp.concatenate(
        [jnp.broadcast_to(
            jnp.sum(e[:, i * F:(i + 1) * F], axis=1, keepdims=True), (BP, F))
         for i in range(8)], axis=1)
    out_ref[...] = o - (m + jnp.log(s))


def _tc3(agg2p, g2p, dinvp, b2_t):
    return pl.pallas_call(
        _tc3_body,
        grid=(NB,),
        in_specs=[
            pl.BlockSpec((BP, 128), lambda i: (i, 0)),
            pl.BlockSpec((BP, 128), lambda i: (NB + i, 0)),
            pl.BlockSpec((BP, 128), lambda i: (i, 0)),
            pl.BlockSpec((BP, 128), lambda i: (i, 0)),
            pl.BlockSpec((1, 128), lambda i: (0, 0)),
        ],
        out_specs=pl.BlockSpec((BP, 128), lambda i: (i, 0)),
        out_shape=jax.ShapeDtypeStruct((NP, 128), jnp.float32),
    )(agg2p, agg2p, g2p, dinvp, b2_t)


def kernel(x, edge_index, fc1_w, fc1_b, conv1_w, conv1_b, conv2_w, conv2_b):
    src = edge_index[0]
    dst = edge_index[1]
    pad_idx = (N + jnp.arange(EPAD - E, dtype=jnp.int32) % (NPAD - N))
    src_p = jnp.concatenate([src, pad_idx]).reshape(TOTBLK, K)
    dst_p = jnp.concatenate([dst, pad_idx]).reshape(TOTBLK, K)
    x_pk = jnp.pad(x, ((0, NPAD - N), (0, 0))).reshape(NP, XW)
    zeros16 = jnp.zeros((NPAD, F), jnp.float32)
    ones16 = jnp.ones((K, F), jnp.float32)
    eye8 = jnp.eye(8, dtype=jnp.float32)
    bd_fc1w = jnp.kron(eye8, fc1_w)
    bd_w1 = jnp.kron(eye8, conv1_w)
    w2_pad = jnp.pad(conv2_w, ((0, 0), (0, F - 6)))
    bd_w2 = jnp.kron(eye8, w2_pad)
    fc1b_t = jnp.tile(fc1_b, 8).reshape(1, 8 * 24)
    b1_t = jnp.tile(conv1_b, 8).reshape(1, 128)
    b2_t = jnp.tile(jnp.pad(conv2_b, ((0, F - 6),)), 8).reshape(1, 128)

    cnt = _sc_degree16(dst_p, ones16, zeros16)
    cntp = jnp.reshape(cnt, (2 * NP, 128))
    g1p, dinvp = _tc1(x_pk, cntp, bd_fc1w, fc1b_t, bd_w1)
    agg1 = _sc_prop16(jnp.reshape(g1p, (NPAD, F)), src_p, dst_p, zeros16)
    agg1p = jnp.reshape(agg1, (2 * NP, 128))
    g2p = _tc2(agg1p, g1p, dinvp, b1_t, bd_w2)
    agg2 = _sc_prop16(jnp.reshape(g2p, (NPAD, F)), src_p, dst_p, zeros16)
    agg2p = jnp.reshape(agg2, (2 * NP, 128))
    outp = _tc3(agg2p, g2p, dinvp, b2_t)
    return jnp.reshape(outp, (NPAD, F))[:N, :6]


# speedup vs baseline: 1.2329x; 1.0607x over previous
import functools

import jax
import jax.numpy as jnp
from jax import lax
from jax.experimental import pallas as pl
from jax.experimental.pallas import tpu as pltpu
from jax.experimental.pallas import tpu_sc as plsc

N = 100000
E = 3200000

NC = 2
NS = 16
NW = NC * NS
K = 128
C = 8
NRB = 8
GAHEAD = 5

NPAD = 100352
RPS = NPAD // NS
NBLK = 792
EPAD = NW * NBLK * K
TOTBLK = EPAD // K
NCHUNK = NBLK // C
SDEPTH = 5

F = 16
NP = NPAD // 8

_mesh = plsc.VectorSubcoreMesh(core_axis_name="c", subcore_axis_name="s")
_sc_params = pltpu.CompilerParams(use_tc_tiling_on_sc=False)


def _make_degree(nrows, w):
  rps = nrows // NS

  @functools.partial(
      pl.kernel,
      out_type=jax.ShapeDtypeStruct((NC * nrows, w), jnp.float32),
      mesh=_mesh,
      scratch_types=[
          pltpu.VMEM((3, C, K), jnp.int32),
          pltpu.VMEM((K, w), jnp.float32),
          pltpu.VMEM_SHARED((nrows, w), jnp.float32),
          pltpu.SemaphoreType.DMA,
          pltpu.SemaphoreType.DMA,
      ],
      compiler_params=_sc_params,
  )
  def _sc_degree(dst_hbm, ones_hbm, zero_hbm, out_hbm, didx, ones_v, acc,
                 isem, ssem):
    cid = lax.axis_index("c")
    sid = lax.axis_index("s")
    wid = sid * NC + cid
    r0 = sid * rps
    pltpu.sync_copy(ones_hbm, ones_v)
    pltpu.sync_copy(zero_hbm.at[pl.ds(r0, rps)], acc.at[pl.ds(r0, rps)])
    plsc.subcore_barrier()

    bb = wid * NBLK
    pltpu.sync_copy(dst_hbm.at[pl.ds(bb, C)], didx.at[0])

    def swait():
        pltpu.make_async_copy(ones_v, acc.at[didx.at[0, 0]], ssem).wait()

    @pl.loop(0, NCHUNK // 3)
    def _(t):
        for q in range(3):
            c = t * 3 + q

            @pl.when(c < NCHUNK - 1)
            def _():
                nb = bb + (c + 1) * C
                pltpu.async_copy(dst_hbm.at[pl.ds(nb, C)],
                                 didx.at[(q + 1) % 3], isem)

            for j in range(C):
                if q == 0 and j < SDEPTH:
                    @pl.when(c > 0)
                    def _():
                        swait()
                else:
                    swait()
                pltpu.async_copy(ones_v, acc.at[didx.at[q, j]], ssem,
                                 add=True)

            @pl.when(c < NCHUNK - 1)
            def _():
                pltpu.make_async_copy(
                    dst_hbm.at[pl.ds(bb, C)], didx.at[(q + 1) % 3],
                    isem).wait()

    for _k in range(SDEPTH):
        swait()
    plsc.subcore_barrier()
    pltpu.sync_copy(acc.at[pl.ds(r0, rps)],
                    out_hbm.at[pl.ds(cid * nrows + r0, rps)])

  return _sc_degree


def _make_propagate(nrows, w):
  rps = nrows // NS

  @functools.partial(
      pl.kernel,
      out_type=jax.ShapeDtypeStruct((NC * nrows, w), jnp.float32),
      mesh=_mesh,
      scratch_types=[
          pltpu.VMEM((3, C, K), jnp.int32),
          pltpu.VMEM((3, C, K), jnp.int32),
      ] + [pltpu.VMEM((K, w), jnp.float32)] * NRB + [
          pltpu.VMEM_SHARED((nrows, w), jnp.float32),
          pltpu.SemaphoreType.DMA,
          pltpu.SemaphoreType.DMA,
          pltpu.SemaphoreType.DMA,
      ],
      compiler_params=_sc_params,
  )
  def _sc_propagate(g_hbm, src_hbm, dst_hbm, zero_hbm, out_hbm,
                    sidx, didx, rows0, rows1, rows2, rows3, rows4, rows5,
                    rows6, rows7, acc, gsem, isem, ssem):
    cid = lax.axis_index("c")
    sid = lax.axis_index("s")
    wid = sid * NC + cid
    r0 = sid * rps
    pltpu.sync_copy(zero_hbm.at[pl.ds(r0, rps)], acc.at[pl.ds(r0, rps)])
    plsc.subcore_barrier()

    bb = wid * NBLK
    pltpu.sync_copy(src_hbm.at[pl.ds(bb, C)], sidx.at[0])
    pltpu.sync_copy(dst_hbm.at[pl.ds(bb, C)], didx.at[0])
    rbufs = (rows0, rows1, rows2, rows3, rows4, rows5, rows6, rows7)
    for j in range(GAHEAD):
        pltpu.async_copy(g_hbm.at[sidx.at[0, j]], rbufs[j], gsem)

    def swait(buf):
        pltpu.make_async_copy(buf, acc.at[didx.at[0, 0]], ssem).wait()

    @pl.loop(0, NCHUNK // 3)
    def _(t):
        for q in range(3):
            c = t * 3 + q

            @pl.when(c < NCHUNK - 1)
            def _():
                nb = bb + (c + 1) * C
                pltpu.async_copy(src_hbm.at[pl.ds(nb, C)],
                                 sidx.at[(q + 1) % 3], isem)
                pltpu.async_copy(dst_hbm.at[pl.ds(nb, C)],
                                 didx.at[(q + 1) % 3], isem)

            for j in range(C):
                fbuf = rbufs[(j + GAHEAD) % NRB]
                if q == 0 and j < NRB - GAHEAD:
                    @pl.when(c > 0)
                    def _():
                        swait(fbuf)
                else:
                    swait(fbuf)
                ahead = j + GAHEAD
                if ahead < C:
                    pltpu.async_copy(g_hbm.at[sidx.at[q, ahead]], fbuf, gsem)
                else:
                    @pl.when(c < NCHUNK - 1)
                    def _():
                        if ahead == C:
                            pltpu.make_async_copy(
                                src_hbm.at[pl.ds(bb, C)],
                                sidx.at[(q + 1) % 3], isem).wait()
                            pltpu.make_async_copy(
                                dst_hbm.at[pl.ds(bb, C)],
                                didx.at[(q + 1) % 3], isem).wait()
                        pltpu.async_copy(
                            g_hbm.at[sidx.at[(q + 1) % 3, ahead - C]], fbuf,
                            gsem)
                cur = rbufs[j]
                pltpu.make_async_copy(g_hbm.at[sidx.at[q, j]], cur,
                                      gsem).wait()
                pltpu.async_copy(cur, acc.at[didx.at[q, j]], ssem, add=True)

    for k in range(NRB - GAHEAD):
        swait(rbufs[(GAHEAD + k) % NRB])
    plsc.subcore_barrier()
    pltpu.sync_copy(acc.at[pl.ds(r0, rps)],
                    out_hbm.at[pl.ds(cid * nrows + r0, rps)])

  return _sc_propagate


_sc_degree16 = _make_degree(NPAD, F)
_sc_prop16 = _make_propagate(NPAD, F)


NB = 7
BP = NP // NB
XW = 8 * 21


def _tc1_body(x_ref, c0_ref, c1_ref, bdw1_ref, b1t_ref, bdwc1_ref,
              g1_ref, dinv_ref):
    dinv = lax.rsqrt(c0_ref[...] + c1_ref[...] + 1.0)
    bdm1 = jnp.dot(bdw1_ref[...], bdwc1_ref[...],
                   preferred_element_type=jnp.float32)
    c1t = jnp.dot(b1t_ref[...], bdwc1_ref[...],
                  preferred_element_type=jnp.float32)
    g1_ref[...] = (jnp.dot(jnp.trunc(x_ref[...]), bdm1,
                           preferred_element_type=jnp.float32) + c1t) * dinv
    dinv_ref[...] = dinv


def _tc1(x_pk, cntp, bd_fc1w, fc1b_t, bd_w1):
    return pl.pallas_call(
        _tc1_body,
        grid=(NB,),
        in_specs=[
            pl.BlockSpec((BP, XW), lambda i: (i, 0)),
            pl.BlockSpec((BP, 128), lambda i: (i, 0)),
            pl.BlockSpec((BP, 128), lambda i: (NB + i, 0)),
            pl.BlockSpec((XW, 8 * 24), lambda i: (0, 0)),
            pl.BlockSpec((1, 8 * 24), lambda i: (0, 0)),
            pl.BlockSpec((8 * 24, 128), lambda i: (0, 0)),
        ],
        out_specs=[
            pl.BlockSpec((BP, 128), lambda i: (i, 0)),
            pl.BlockSpec((BP, 128), lambda i: (i, 0)),
        ],
        out_shape=[
            jax.ShapeDtypeStruct((NP, 128), jnp.float32),
            jax.ShapeDtypeStruct((NP, 128), jnp.float32),
        ],
    )(x_pk, cntp, cntp, bd_fc1w, fc1b_t, bd_w1)


def _tc2_body(a0_ref, a1_ref, g1_ref, dinv_ref, b1t_ref, bdw2_ref, g2_ref):
    d = dinv_ref[...]
    h1 = d * (a0_ref[...] + a1_ref[...] + g1_ref[...]) + b1t_ref[...]
    h1 = jnp.maximum(h1, 0.0)
    g2_ref[...] = jnp.dot(h1, bdw2_ref[...],
                          preferred_element_type=jnp.float32) * d


def _tc2(agg1p, g1p, dinvp, b1_t, bd_w2):
    return pl.pallas_call(
        _tc2_body,
        grid=(NB,),
        in_specs=[
            pl.BlockSpec((BP, 128), lambda i: (i, 0)),
            pl.BlockSpec((BP, 128), lambda i: (NB + i, 0)),
            pl.BlockSpec((BP, 128), lambda i: (i, 0)),
            pl.BlockSpec((BP, 128), lambda i: (i, 0)),
            pl.BlockSpec((1, 128), lambda i: (0, 0)),
            pl.BlockSpec((128, 128), lambda i: (0, 0)),
        ],
        out_specs=pl.BlockSpec((BP, 128), lambda i: (i, 0)),
        out_shape=jax.ShapeDtypeStruct((NP, 128), jnp.float32),
    )(agg1p, agg1p, g1p, dinvp, b1_t, bd_w2)


def _tc3_body(a0_ref, a1_ref, g2_ref, dinv_ref, b2t_ref, out_ref):
    o = dinv_ref[...] * (a0_ref[...] + a1_ref[...] + g2_ref[...]) + b2t_ref[...]
    mask = lax.broadcasted_iota(jnp.int32, (1, 128), 1) % F < 6
    z = jnp.where(mask, o, -jnp.inf)
    m = jnp.concatenate(
        [jnp.broadcast_to(
            jnp.max(z[:, i * F:(i + 1) * F], axis=1, keepdims=True), (BP, F))
         for i in range(8)], axis=1)
    e = jnp.where(mask, jnp.exp(z - m), 0.0)
    s = jnp.concatenate(
        [jnp.broadcast_to(
            jnp.sum(e[:, i * F:(i + 1) * F], axis=1, keepdims=True), (BP, F))
         for i in range(8)], axis=1)
    out_ref[...] = o - (m + jnp.log(s))


def _tc3(agg2p, g2p, dinvp, b2_t):
    return pl.pallas_call(
        _tc3_body,
        grid=(NB,),
        in_specs=[
            pl.BlockSpec((BP, 128), lambda i: (i, 0)),
            pl.BlockSpec((BP, 128), lambda i: (NB + i, 0)),
            pl.BlockSpec((BP, 128), lambda i: (i, 0)),
            pl.BlockSpec((BP, 128), lambda i: (i, 0)),
            pl.BlockSpec((1, 128), lambda i: (0, 0)),
        ],
        out_specs=pl.BlockSpec((BP, 128), lambda i: (i, 0)),
        out_shape=jax.ShapeDtypeStruct((NP, 128), jnp.float32),
    )(agg2p, agg2p, g2p, dinvp, b2_t)


def kernel(x, edge_index, fc1_w, fc1_b, conv1_w, conv1_b, conv2_w, conv2_b):
    src = edge_index[0]
    dst = edge_index[1]
    pad_idx = (N + jnp.arange(EPAD - E, dtype=jnp.int32) % (NPAD - N))
    src_p = jnp.concatenate([src, pad_idx]).reshape(TOTBLK, K)
    dst_p = jnp.concatenate([dst, pad_idx]).reshape(TOTBLK, K)
    x_pk = jnp.pad(x, ((0, NPAD - N), (0, 0))).reshape(NP, XW)
    zeros16 = jnp.zeros((NPAD, F), jnp.float32)
    ones16 = jnp.ones((K, F), jnp.float32)
    eye8 = jnp.eye(8, dtype=jnp.float32)
    bd_fc1w = jnp.kron(eye8, fc1_w)
    bd_w1 = jnp.kron(eye8, conv1_w)
    w2_pad = jnp.pad(conv2_w, ((0, 0), (0, F - 6)))
    bd_w2 = jnp.kron(eye8, w2_pad)
    fc1b_t = jnp.tile(fc1_b, 8).reshape(1, 8 * 24)
    b1_t = jnp.tile(conv1_b, 8).reshape(1, 128)
    b2_t = jnp.tile(jnp.pad(conv2_b, ((0, F - 6),)), 8).reshape(1, 128)

    cnt = _sc_degree16(dst_p, ones16, zeros16)
    cntp = jnp.reshape(cnt, (2 * NP, 128))
    g1p, dinvp = _tc1(x_pk, cntp, bd_fc1w, fc1b_t, bd_w1)
    agg1 = _sc_prop16(jnp.reshape(g1p, (NPAD, F)), src_p, dst_p, zeros16)
    agg1p = jnp.reshape(agg1, (2 * NP, 128))
    g2p = _tc2(agg1p, g1p, dinvp, b1_t, bd_w2)
    agg2 = _sc_prop16(jnp.reshape(g2p, (NPAD, F)), src_p, dst_p, zeros16)
    agg2p = jnp.reshape(agg2, (2 * NP, 128))
    outp = _tc3(agg2p, g2p, dinvp, b2_t)
    return jnp.reshape(outp, (NPAD, F))[:N, :6]


# speedup vs baseline: 1.2672x; 1.0278x over previous
import functools

import jax
import jax.numpy as jnp
from jax import lax
from jax.experimental import pallas as pl
from jax.experimental.pallas import tpu as pltpu
from jax.experimental.pallas import tpu_sc as plsc

N = 100000
E = 3200000

NC = 2
NS = 16
NW = NC * NS
K = 128
C = 8
NRB = 8
GAHEAD = 6

NPAD = 100352
RPS = NPAD // NS
NBLK = 792
EPAD = NW * NBLK * K
TOTBLK = EPAD // K
NCHUNK = NBLK // C
SDEPTH = 5

F = 16
NP = NPAD // 8

_mesh = plsc.VectorSubcoreMesh(core_axis_name="c", subcore_axis_name="s")
_sc_params = pltpu.CompilerParams(use_tc_tiling_on_sc=False)


def _make_degree(nrows, w):
  rps = nrows // NS

  @functools.partial(
      pl.kernel,
      out_type=jax.ShapeDtypeStruct((NC * nrows, w), jnp.float32),
      mesh=_mesh,
      scratch_types=[
          pltpu.VMEM((3, C, K), jnp.int32),
          pltpu.VMEM((K, w), jnp.float32),
          pltpu.VMEM_SHARED((nrows, w), jnp.float32),
          pltpu.SemaphoreType.DMA,
          pltpu.SemaphoreType.DMA,
      ],
      compiler_params=_sc_params,
  )
  def _sc_degree(dst_hbm, ones_hbm, zero_hbm, out_hbm, didx, ones_v, acc,
                 isem, ssem):
    cid = lax.axis_index("c")
    sid = lax.axis_index("s")
    wid = sid * NC + cid
    r0 = sid * rps
    pltpu.sync_copy(ones_hbm, ones_v)
    pltpu.sync_copy(zero_hbm.at[pl.ds(r0, rps)], acc.at[pl.ds(r0, rps)])
    plsc.subcore_barrier()

    bb = wid * NBLK
    pltpu.sync_copy(dst_hbm.at[pl.ds(bb, C)], didx.at[0])

    def swait():
        pltpu.make_async_copy(ones_v, acc.at[didx.at[0, 0]], ssem).wait()

    @pl.loop(0, NCHUNK // 3)
    def _(t):
        for q in range(3):
            c = t * 3 + q

            @pl.when(c < NCHUNK - 1)
            def _():
                nb = bb + (c + 1) * C
                pltpu.async_copy(dst_hbm.at[pl.ds(nb, C)],
                                 didx.at[(q + 1) % 3], isem)

            for j in range(C):
                if q == 0 and j < SDEPTH:
                    @pl.when(c > 0)
                    def _():
                        swait()
                else:
                    swait()
                pltpu.async_copy(ones_v, acc.at[didx.at[q, j]], ssem,
                                 add=True)

            @pl.when(c < NCHUNK - 1)
            def _():
                pltpu.make_async_copy(
                    dst_hbm.at[pl.ds(bb, C)], didx.at[(q + 1) % 3],
                    isem).wait()

    for _k in range(SDEPTH):
        swait()
    plsc.subcore_barrier()
    pltpu.sync_copy(acc.at[pl.ds(r0, rps)],
                    out_hbm.at[pl.ds(cid * nrows + r0, rps)])

  return _sc_degree


def _make_propagate(nrows, w):
  rps = nrows // NS

  @functools.partial(
      pl.kernel,
      out_type=jax.ShapeDtypeStruct((NC * nrows, w), jnp.float32),
      mesh=_mesh,
      scratch_types=[
          pltpu.VMEM((3, C, K), jnp.int32),
          pltpu.VMEM((3, C, K), jnp.int32),
      ] + [pltpu.VMEM((K, w), jnp.float32)] * NRB + [
          pltpu.VMEM_SHARED((nrows, w), jnp.float32),
          pltpu.SemaphoreType.DMA,
          pltpu.SemaphoreType.DMA,
          pltpu.SemaphoreType.DMA,
      ],
      compiler_params=_sc_params,
  )
  def _sc_propagate(g_hbm, src_hbm, dst_hbm, zero_hbm, out_hbm,
                    sidx, didx, rows0, rows1, rows2, rows3, rows4, rows5,
                    rows6, rows7, acc, gsem, isem, ssem):
    cid = lax.axis_index("c")
    sid = lax.axis_index("s")
    wid = sid * NC + cid
    r0 = sid * rps
    pltpu.sync_copy(zero_hbm.at[pl.ds(r0, rps)], acc.at[pl.ds(r0, rps)])
    plsc.subcore_barrier()

    bb = wid * NBLK
    pltpu.sync_copy(src_hbm.at[pl.ds(bb, C)], sidx.at[0])
    pltpu.sync_copy(dst_hbm.at[pl.ds(bb, C)], didx.at[0])
    rbufs = (rows0, rows1, rows2, rows3, rows4, rows5, rows6, rows7)
    for j in range(GAHEAD):
        pltpu.async_copy(g_hbm.at[sidx.at[0, j]], rbufs[j], gsem)

    def swait(buf):
        pltpu.make_async_copy(buf, acc.at[didx.at[0, 0]], ssem).wait()

    @pl.loop(0, NCHUNK // 3)
    def _(t):
        for q in range(3):
            c = t * 3 + q

            @pl.when(c < NCHUNK - 1)
            def _():
                nb = bb + (c + 1) * C
                pltpu.async_copy(src_hbm.at[pl.ds(nb, C)],
                                 sidx.at[(q + 1) % 3], isem)
                pltpu.async_copy(dst_hbm.at[pl.ds(nb, C)],
                                 didx.at[(q + 1) % 3], isem)

            for j in range(C):
                fbuf = rbufs[(j + GAHEAD) % NRB]
                if q == 0 and j < NRB - GAHEAD:
                    @pl.when(c > 0)
                    def _():
                        swait(fbuf)
                else:
                    swait(fbuf)
                ahead = j + GAHEAD
                if ahead < C:
                    pltpu.async_copy(g_hbm.at[sidx.at[q, ahead]], fbuf, gsem)
                else:
                    @pl.when(c < NCHUNK - 1)
                    def _():
                        if ahead == C:
                            pltpu.make_async_copy(
                                src_hbm.at[pl.ds(bb, C)],
                                sidx.at[(q + 1) % 3], isem).wait()
                            pltpu.make_async_copy(
                                dst_hbm.at[pl.ds(bb, C)],
                                didx.at[(q + 1) % 3], isem).wait()
                        pltpu.async_copy(
                            g_hbm.at[sidx.at[(q + 1) % 3, ahead - C]], fbuf,
                            gsem)
                cur = rbufs[j]
                pltpu.make_async_copy(g_hbm.at[sidx.at[q, j]], cur,
                                      gsem).wait()
                pltpu.async_copy(cur, acc.at[didx.at[q, j]], ssem, add=True)

    for k in range(NRB - GAHEAD):
        swait(rbufs[(GAHEAD + k) % NRB])
    plsc.subcore_barrier()
    pltpu.sync_copy(acc.at[pl.ds(r0, rps)],
                    out_hbm.at[pl.ds(cid * nrows + r0, rps)])

  return _sc_propagate


_sc_degree16 = _make_degree(NPAD, F)
_sc_prop16 = _make_propagate(NPAD, F)


NB = 7
BP = NP // NB
XW = 8 * 21


def _tc1_body(x_ref, c0_ref, c1_ref, bdw1_ref, b1t_ref, bdwc1_ref,
              g1_ref, dinv_ref):
    dinv = lax.rsqrt(c0_ref[...] + c1_ref[...] + 1.0)
    bdm1 = jnp.dot(bdw1_ref[...], bdwc1_ref[...],
                   preferred_element_type=jnp.float32)
    c1t = jnp.dot(b1t_ref[...], bdwc1_ref[...],
                  preferred_element_type=jnp.float32)
    g1_ref[...] = (jnp.dot(jnp.trunc(x_ref[...]), bdm1,
                           preferred_element_type=jnp.float32) + c1t) * dinv
    dinv_ref[...] = dinv


def _tc1(x_pk, cntp, bd_fc1w, fc1b_t, bd_w1):
    return pl.pallas_call(
        _tc1_body,
        grid=(NB,),
        in_specs=[
            pl.BlockSpec((BP, XW), lambda i: (i, 0)),
            pl.BlockSpec((BP, 128), lambda i: (i, 0)),
            pl.BlockSpec((BP, 128), lambda i: (NB + i, 0)),
            pl.BlockSpec((XW, 8 * 24), lambda i: (0, 0)),
            pl.BlockSpec((1, 8 * 24), lambda i: (0, 0)),
            pl.BlockSpec((8 * 24, 128), lambda i: (0, 0)),
        ],
        out_specs=[
            pl.BlockSpec((BP, 128), lambda i: (i, 0)),
            pl.BlockSpec((BP, 128), lambda i: (i, 0)),
        ],
        out_shape=[
            jax.ShapeDtypeStruct((NP, 128), jnp.float32),
            jax.ShapeDtypeStruct((NP, 128), jnp.float32),
        ],
    )(x_pk, cntp, cntp, bd_fc1w, fc1b_t, bd_w1)


def _tc2_body(a0_ref, a1_ref, g1_ref, dinv_ref, b1t_ref, bdw2_ref, g2_ref):
    d = dinv_ref[...]
    h1 = d * (a0_ref[...] + a1_ref[...] + g1_ref[...]) + b1t_ref[...]
    h1 = jnp.maximum(h1, 0.0)
    g2_ref[...] = jnp.dot(h1, bdw2_ref[...],
                          preferred_element_type=jnp.float32) * d


def _tc2(agg1p, g1p, dinvp, b1_t, bd_w2):
    return pl.pallas_call(
        _tc2_body,
        grid=(NB,),
        in_specs=[
            pl.BlockSpec((BP, 128), lambda i: (i, 0)),
            pl.BlockSpec((BP, 128), lambda i: (NB + i, 0)),
            pl.BlockSpec((BP, 128), lambda i: (i, 0)),
            pl.BlockSpec((BP, 128), lambda i: (i, 0)),
            pl.BlockSpec((1, 128), lambda i: (0, 0)),
            pl.BlockSpec((128, 128), lambda i: (0, 0)),
        ],
        out_specs=pl.BlockSpec((BP, 128), lambda i: (i, 0)),
        out_shape=jax.ShapeDtypeStruct((NP, 128), jnp.float32),
    )(agg1p, agg1p, g1p, dinvp, b1_t, bd_w2)


def _tc3_body(a0_ref, a1_ref, g2_ref, dinv_ref, b2t_ref, out_ref):
    o = dinv_ref[...] * (a0_ref[...] + a1_ref[...] + g2_ref[...]) + b2t_ref[...]
    mask = lax.broadcasted_iota(jnp.int32, (1, 128), 1) % F < 6
    z = jnp.where(mask, o, -jnp.inf)
    m = jnp.concatenate(
        [jnp.broadcast_to(
            jnp.max(z[:, i * F:(i + 1) * F], axis=1, keepdims=True), (BP, F))
         for i in range(8)], axis=1)
    e = jnp.where(mask, jnp.exp(z - m), 0.0)
    s = jnp.concatenate(
        [jnp.broadcast_to(
            jnp.sum(e[:, i * F:(i + 1) * F], axis=1, keepdims=True), (BP, F))
         for i in range(8)], axis=1)
    out_ref[...] = o - (m + jnp.log(s))


def _tc3(agg2p, g2p, dinvp, b2_t):
    return pl.pallas_call(
        _tc3_body,
        grid=(NB,),
        in_specs=[
            pl.BlockSpec((BP, 128), lambda i: (i, 0)),
            pl.BlockSpec((BP, 128), lambda i: (NB + i, 0)),
            pl.BlockSpec((BP, 128), lambda i: (i, 0)),
            pl.BlockSpec((BP, 128), lambda i: (i, 0)),
            pl.BlockSpec((1, 128), lambda i: (0, 0)),
        ],
        out_specs=pl.BlockSpec((BP, 128), lambda i: (i, 0)),
        out_shape=jax.ShapeDtypeStruct((NP, 128), jnp.float32),
    )(agg2p, agg2p, g2p, dinvp, b2_t)


def kernel(x, edge_index, fc1_w, fc1_b, conv1_w, conv1_b, conv2_w, conv2_b):
    src = edge_index[0]
    dst = edge_index[1]
    pad_idx = (N + jnp.arange(EPAD - E, dtype=jnp.int32) % (NPAD - N))
    src_p = jnp.concatenate([src, pad_idx]).reshape(TOTBLK, K)
    dst_p = jnp.concatenate([dst, pad_idx]).reshape(TOTBLK, K)
    x_pk = jnp.pad(x, ((0, NPAD - N), (0, 0))).reshape(NP, XW)
    zeros16 = jnp.zeros((NPAD, F), jnp.float32)
    ones16 = jnp.ones((K, F), jnp.float32)
    eye8 = jnp.eye(8, dtype=jnp.float32)
    bd_fc1w = jnp.kron(eye8, fc1_w)
    bd_w1 = jnp.kron(eye8, conv1_w)
    w2_pad = jnp.pad(conv2_w, ((0, 0), (0, F - 6)))
    bd_w2 = jnp.kron(eye8, w2_pad)
    fc1b_t = jnp.tile(fc1_b, 8).reshape(1, 8 * 24)
    b1_t = jnp.tile(conv1_b, 8).reshape(1, 128)
    b2_t = jnp.tile(jnp.pad(conv2_b, ((0, F - 6),)), 8).reshape(1, 128)

    cnt = _sc_degree16(dst_p, ones16, zeros16)
    cntp = jnp.reshape(cnt, (2 * NP, 128))
    g1p, dinvp = _tc1(x_pk, cntp, bd_fc1w, fc1b_t, bd_w1)
    agg1 = _sc_prop16(jnp.reshape(g1p, (NPAD, F)), src_p, dst_p, zeros16)
    agg1p = jnp.reshape(agg1, (2 * NP, 128))
    g2p = _tc2(agg1p, g1p, dinvp, b1_t, bd_w2)
    agg2 = _sc_prop16(jnp.reshape(g2p, (NPAD, F)), src_p, dst_p, zeros16)
    agg2p = jnp.reshape(agg2, (2 * NP, 128))
    outp = _tc3(agg2p, g2p, dinvp, b2_t)
    return jnp.reshape(outp, (NPAD, F))[:N, :6]
